# Initial kernel scaffold; baseline (speedup 1.0000x reference)
import jax
import jax.numpy as jnp
from jax.experimental import pallas as pl


def kernel(hidden_states, gate_w, w_gate_up, w_down, sw_gate_up, sw_down):
    raise NotImplementedError("write your pallas kernel here")


# speedup vs baseline: 3.9127x; 3.9127x over previous
import functools

import jax
import jax.numpy as jnp
from jax.experimental import pallas as pl
from jax.experimental.pallas import tpu as pltpu

T = 2048
D = 1024
E = 64
I = 512
B = 128
G = 80
R = G * B
TPAD = 2304


def _router_shared_body(h_ref, gwp_ref, swgu_ref, swd_ref, init_ref, eid_ref):
    h = h_ref[...]
    logits = jnp.dot(h, gwp_ref[...])
    rl = logits[:, :E]
    mx = jnp.max(rl, axis=1, keepdims=True)
    col = jax.lax.broadcasted_iota(jnp.int32, rl.shape, 1)
    cand = jnp.where(rl == mx, col, E)
    eid = jnp.min(cand, axis=1)
    sg = jax.nn.sigmoid(logits[:, E])

    gu = jnp.dot(h, swgu_ref[...])
    g = gu[:, :I]
    u = gu[:, I:]
    hh = g * jax.nn.sigmoid(g) * u
    sh = jnp.dot(hh, swd_ref[...])
    init_ref[...] = sg[:, None] * sh
    eid_ref[...] = eid.reshape(1, 1, 256)


def _router_shared(h, gate_w_pad, sw_gate_up, sw_down):
    return pl.pallas_call(
        _router_shared_body,
        grid=(T // 256,),
        in_specs=[
            pl.BlockSpec((256, D), lambda g: (g, 0)),
            pl.BlockSpec((D, 128), lambda g: (0, 0)),
            pl.BlockSpec((D, 2 * I), lambda g: (0, 0)),
            pl.BlockSpec((I, D), lambda g: (0, 0)),
        ],
        out_specs=[
            pl.BlockSpec((256, D), lambda g: (g, 0)),
            pl.BlockSpec((1, 1, 256), lambda g: (g, 0, 0)),
        ],
        out_shape=[
            jax.ShapeDtypeStruct((T, D), jnp.float32),
            jax.ShapeDtypeStruct((T // 256, 1, 256), jnp.int32),
        ],
    )(h, gate_w_pad, sw_gate_up, sw_down)


def _expert_body(be_ref, x_ref, wgu_ref, wd_ref, o_ref):
    x = x_ref[...]
    gu = jnp.dot(x, wgu_ref[0])
    g = gu[:, :I]
    u = gu[:, I:]
    hh = g * jax.nn.sigmoid(g) * u
    o_ref[...] = jnp.dot(hh, wd_ref[0])


def _expert_matmul(block_expert, x_sorted, w_gate_up, w_down):
    grid_spec = pltpu.PrefetchScalarGridSpec(
        num_scalar_prefetch=1,
        grid=(G,),
        in_specs=[
            pl.BlockSpec((B, D), lambda g, be: (g, 0)),
            pl.BlockSpec((1, D, 2 * I), lambda g, be: (be[g], 0, 0)),
            pl.BlockSpec((1, I, D), lambda g, be: (be[g], 0, 0)),
        ],
        out_specs=pl.BlockSpec((B, D), lambda g, be: (g, 0)),
    )
    return pl.pallas_call(
        _expert_body,
        grid_spec=grid_spec,
        out_shape=jax.ShapeDtypeStruct((R, D), jnp.float32),
    )(block_expert, x_sorted, w_gate_up, w_down)


def _add_body(a_ref, b_ref, o_ref):
    o_ref[...] = a_ref[...] + b_ref[...]


def _combine(init, scattered):
    return pl.pallas_call(
        _add_body,
        grid=(T // 256,),
        in_specs=[
            pl.BlockSpec((256, D), lambda g: (g, 0)),
            pl.BlockSpec((256, D), lambda g: (g, 0)),
        ],
        out_specs=pl.BlockSpec((256, D), lambda g: (g, 0)),
        out_shape=jax.ShapeDtypeStruct((T, D), jnp.float32),
    )(init, scattered)


def _dispatch_indices(eids_flat):
    counts = jnp.zeros((E,), jnp.int32).at[eids_flat].add(1)
    nb = (counts + B - 1) // B
    bs = jnp.cumsum(nb) - nb
    tok_cum = jnp.cumsum(counts) - counts
    order = jnp.argsort(eids_flat, stable=True)
    e_sorted = eids_flat[order]
    j = jnp.arange(T, dtype=jnp.int32)
    pos = bs[e_sorted] * B + (j - tok_cum[e_sorted])
    src_idx = jnp.zeros((R,), jnp.int32).at[pos].set(order.astype(jnp.int32))
    real = jnp.zeros((R,), jnp.bool_).at[pos].set(True)
    dst_idx = jnp.where(real, src_idx,
                        T + (jnp.arange(R, dtype=jnp.int32) % 64))
    return src_idx, dst_idx


def kernel(hidden_states, gate_w, w_gate_up, w_down, sw_gate_up, sw_down):
    gate_w_pad = jnp.pad(gate_w, ((0, 0), (0, 128 - (E + 1))))
    init, eids = _router_shared(hidden_states, gate_w_pad, sw_gate_up, sw_down)
    eids_flat = eids.reshape(T)

    src_idx, dst_idx = _dispatch_indices(eids_flat)
    x_sorted = hidden_states[src_idx]
    block_expert = eids_flat[src_idx[::B]]

    routed = _expert_matmul(block_expert, x_sorted, w_gate_up, w_down)

    scattered = jnp.zeros((TPAD, D), jnp.float32).at[dst_idx].set(routed)
    return _combine(init, scattered[:T])
